# baseline (speedup 1.0000x reference)
import functools

import jax
import jax.numpy as jnp
from jax import lax
from jax.experimental import pallas as pl
from jax.experimental.pallas import tpu as pltpu
from jax.experimental.pallas import tpu_sc as plsc

_N_FIELDS = 26
_VOCAB = 1000
_EMB = 128
_BATCH = 4096
_N_NUM = 13

_NW = 32
_ROWS = _BATCH // _NW
_LANES = 16
_NB = 5
_GROUPS = (7, 7, 7, 5)


def _make_body(f0, nf, with_num):
    def _body(xt_hbm, xnum_hbm, table_hbm, out_hbm, idx_v, rows_v, xn_v, gsem, wsem, nsem):
        wid = lax.axis_index("s") * 2 + lax.axis_index("c")
        b0 = wid * _ROWS
        ncopy = None
        if with_num:
            pltpu.sync_copy(xnum_hbm.at[pl.ds(b0, _ROWS)], xn_v)
            ncopy = pltpu.make_async_copy(
                xn_v, out_hbm.at[pl.ds(b0, _ROWS), pl.ds(nf * _EMB, _N_NUM)], nsem
            )
            ncopy.start()
        pltpu.sync_copy(xt_hbm.at[:, pl.ds(b0, _ROWS)], idx_v)
        for k in range(nf):
            f = f0 + k
            if f:
                off = f * _VOCAB
                for j in range(_ROWS // _LANES):
                    sl = pl.ds(j * _LANES, _LANES)
                    idx_v[f, sl] = idx_v[f, sl] + off

        gd = [None] * nf
        wd = [None] * nf

        def gstart(k):
            s = k % _NB
            gd[k] = pltpu.make_async_copy(
                table_hbm.at[idx_v.at[f0 + k]], rows_v.at[s], gsem.at[s]
            )
            gd[k].start()

        def wstart(k):
            s = k % _NB
            wd[k] = pltpu.make_async_copy(
                rows_v.at[s],
                out_hbm.at[pl.ds(b0, _ROWS), pl.ds(k * _EMB, _EMB)],
                wsem.at[s],
            )
            wd[k].start()

        for k in range(min(_NB - 1, nf)):
            gstart(k)
        for k in range(nf):
            gd[k].wait()
            wstart(k)
            wd[k].wait()
            nk = k + _NB - 1
            if nk < nf:
                gstart(nk)
        if ncopy is not None:
            ncopy.wait()

    return _body


@jax.jit
def kernel(x_cat, x_num, tables):
    xt = x_cat.astype(jnp.int32).T
    table = tables.reshape(_N_FIELDS * _VOCAB, _EMB)
    outs = []
    f0 = 0
    for gi, nf in enumerate(_GROUPS):
        with_num = gi == len(_GROUPS) - 1
        width = nf * _EMB + (_N_NUM if with_num else 0)
        run = functools.partial(
            pl.kernel,
            out_type=jax.ShapeDtypeStruct((_BATCH, width), jnp.float32),
            mesh=plsc.VectorSubcoreMesh(core_axis_name="c", subcore_axis_name="s"),
            compiler_params=pltpu.CompilerParams(use_tc_tiling_on_sc=True),
            scratch_types=[
                pltpu.VMEM((_N_FIELDS, _ROWS), jnp.int32),
                pltpu.VMEM((_NB, _ROWS, _EMB), jnp.float32),
                pltpu.VMEM((_ROWS, _N_NUM), jnp.float32),
                pltpu.SemaphoreType.DMA((_NB,)),
                pltpu.SemaphoreType.DMA((_NB,)),
                pltpu.SemaphoreType.DMA,
            ],
        )(_make_body(f0, nf, with_num))
        outs.append(run(xt, x_num, table))
        f0 += nf
    return jnp.concatenate(outs, axis=1)


# speedup vs baseline: 1.0237x; 1.0237x over previous
import functools

import jax
import jax.numpy as jnp
from jax import lax
from jax.experimental import pallas as pl
from jax.experimental.pallas import tpu as pltpu
from jax.experimental.pallas import tpu_sc as plsc

_N_FIELDS = 26
_VOCAB = 1000
_EMB = 128
_BATCH = 4096
_N_NUM = 13

_NW = 32
_ROWS = _BATCH // _NW
_LANES = 16
_NB = 5
_GROUPS = (7, 7, 7, 5)


def _make_body(f0, nf, with_num):
    def _body(xt_hbm, xnum_hbm, table_hbm, out_hbm, idx_v, rows_v, xn_v, gsem, wsem, nsem):
        wid = lax.axis_index("s") * 2 + lax.axis_index("c")
        b0 = wid * _ROWS
        ncopy = None
        if with_num:
            pltpu.sync_copy(xnum_hbm.at[pl.ds(b0, _ROWS)], xn_v)
            ncopy = pltpu.make_async_copy(
                xn_v, out_hbm.at[pl.ds(b0, _ROWS), pl.ds(nf * _EMB, _N_NUM)], nsem
            )
            ncopy.start()
        pltpu.sync_copy(xt_hbm.at[:, pl.ds(b0, _ROWS)], idx_v)
        for k in range(nf):
            f = f0 + k
            if f:
                off = f * _VOCAB
                for j in range(_ROWS // _LANES):
                    sl = pl.ds(j * _LANES, _LANES)
                    idx_v[f, sl] = idx_v[f, sl] + off

        gd = [None] * nf
        wd = [None] * nf

        def gstart(k):
            s = k % _NB
            gd[k] = pltpu.make_async_copy(
                table_hbm.at[idx_v.at[f0 + k]], rows_v.at[s], gsem.at[s]
            )
            gd[k].start()

        def wstart(k):
            s = k % _NB
            wd[k] = pltpu.make_async_copy(
                rows_v.at[s],
                out_hbm.at[pl.ds(b0, _ROWS), pl.ds(k * _EMB, _EMB)],
                wsem.at[s],
            )
            wd[k].start()

        for k in range(min(_NB - 1, nf)):
            gstart(k)
        for k in range(nf):
            gd[k].wait()
            wstart(k)
            wd[k].wait()
            nk = k + _NB - 1
            if nk < nf:
                gstart(nk)
        if ncopy is not None:
            ncopy.wait()

    return _body


@jax.jit
def kernel(x_cat, x_num, tables):
    xt = x_cat.astype(jnp.int32).T
    table = tables.reshape(_N_FIELDS * _VOCAB, _EMB)
    outs = []
    f0 = 0
    for gi, nf in enumerate(_GROUPS):
        with_num = gi == len(_GROUPS) - 1
        width = nf * _EMB + (_N_NUM if with_num else 0)
        run = functools.partial(
            pl.kernel,
            out_type=jax.ShapeDtypeStruct((_BATCH, width), jnp.float32),
            mesh=plsc.VectorSubcoreMesh(core_axis_name="c", subcore_axis_name="s"),
            compiler_params=pltpu.CompilerParams(use_tc_tiling_on_sc=True),
            scratch_types=[
                pltpu.VMEM((_N_FIELDS, _ROWS), jnp.int32),
                pltpu.VMEM((_NB, _ROWS, _EMB), jnp.float32),
                pltpu.VMEM((_ROWS, _N_NUM), jnp.float32),
                pltpu.SemaphoreType.DMA((_NB,)),
                pltpu.SemaphoreType.DMA((_NB,)),
                pltpu.SemaphoreType.DMA,
            ],
        )(_make_body(f0, nf, with_num))
        outs.append(run(xt, x_num, table))
        f0 += nf
    res = jnp.zeros((_BATCH, _N_FIELDS * _EMB + _N_NUM), jnp.float32)
    f0 = 0
    for o in outs:
        res = lax.dynamic_update_slice(res, o, (0, f0 * _EMB))
        f0 += o.shape[1] // _EMB
    return res


# speedup vs baseline: 1.4369x; 1.4037x over previous
import functools

import jax
import jax.numpy as jnp
from jax import lax
from jax.experimental import pallas as pl
from jax.experimental.pallas import tpu as pltpu
from jax.experimental.pallas import tpu_sc as plsc

_N_FIELDS = 26
_VOCAB = 1000
_EMB = 128
_BATCH = 4096
_N_NUM = 13
_OUT_W = _N_FIELDS * _EMB + _N_NUM

_NW = 32
_ROWS = _BATCH // _NW
_LANES = 16
_NB = 5


def _body(xt_hbm, xnum_hbm, table_hbm, out_hbm, idx_v, rows_v, xn_v, gsem, wsem, nsem):
    wid = lax.axis_index("s") * 2 + lax.axis_index("c")
    b0 = wid * _ROWS
    pltpu.sync_copy(xnum_hbm.at[pl.ds(b0, _ROWS)], xn_v)
    ncopy = pltpu.make_async_copy(
        xn_v, out_hbm.at[pl.ds(b0, _ROWS), pl.ds(_N_FIELDS * _EMB, _N_NUM)], nsem
    )
    ncopy.start()
    pltpu.sync_copy(xt_hbm.at[:, pl.ds(b0, _ROWS)], idx_v)
    for f in range(1, _N_FIELDS):
        off = f * _VOCAB
        for j in range(_ROWS // _LANES):
            sl = pl.ds(j * _LANES, _LANES)
            idx_v[f, sl] = idx_v[f, sl] + off

    gd = [None] * _N_FIELDS
    wd = [None] * _N_FIELDS

    def gstart(f):
        s = f % _NB
        gd[f] = pltpu.make_async_copy(
            table_hbm.at[idx_v.at[f]], rows_v.at[s], gsem.at[s]
        )
        gd[f].start()

    def wstart(f):
        s = f % _NB
        wd[f] = pltpu.make_async_copy(
            rows_v.at[s],
            out_hbm.at[pl.ds(b0, _ROWS), pl.ds(f * _EMB, _EMB)],
            wsem.at[s],
        )
        wd[f].start()

    for f in range(_NB - 1):
        gstart(f)
    for f in range(_N_FIELDS):
        gd[f].wait()
        wstart(f)
        wd[f].wait()
        nf = f + _NB - 1
        if nf < _N_FIELDS:
            gstart(nf)
    ncopy.wait()


@jax.jit
def kernel(x_cat, x_num, tables):
    xt = x_cat.astype(jnp.int32).T
    table = tables.reshape(_N_FIELDS * _VOCAB, _EMB)
    run = functools.partial(
        pl.kernel,
        out_type=jax.ShapeDtypeStruct((_BATCH, _OUT_W), jnp.float32),
        mesh=plsc.VectorSubcoreMesh(core_axis_name="c", subcore_axis_name="s"),
        compiler_params=pltpu.CompilerParams(use_tc_tiling_on_sc=True),
        scratch_types=[
            pltpu.VMEM((_N_FIELDS, _ROWS), jnp.int32),
            pltpu.VMEM((_NB, _ROWS, _EMB), jnp.float32),
            pltpu.VMEM((_ROWS, _N_NUM), jnp.float32),
            pltpu.SemaphoreType.DMA((_NB,)),
            pltpu.SemaphoreType.DMA((_NB,)),
            pltpu.SemaphoreType.DMA,
        ],
    )(_body)
    return run(xt, x_num, table)
